# baseline (speedup 1.0000x reference)
import functools

import jax
import jax.numpy as jnp
from jax import lax
from jax.experimental import pallas as pl
from jax.experimental.pallas import tpu as pltpu
from jax.experimental.pallas import tpu_sc as plsc


def kernel(patches, table):
    B, P, D = patches.shape
    E = table.shape[1]
    info = plsc.get_sparse_core_info()
    NC, NS = info.num_cores, info.num_subcores
    NW = NC * NS
    nb = B // NW
    mesh = plsc.VectorSubcoreMesh(core_axis_name="c", subcore_axis_name="s")

    @functools.partial(
        pl.kernel,
        mesh=mesh,
        out_type=jax.ShapeDtypeStruct((B, P, D + E), patches.dtype),
        scratch_types=[pltpu.SemaphoreType.DMA],
    )
    def k(p_hbm, t_hbm, o_hbm, sem):
        wid = lax.axis_index("s") * NC + lax.axis_index("c")
        copies = []
        for i in range(nb):
            b = wid * nb + i
            copies.append(
                pltpu.async_copy(p_hbm.at[b], o_hbm.at[b, :, pl.ds(0, D)], sem))
            copies.append(
                pltpu.async_copy(t_hbm, o_hbm.at[b, :, pl.ds(D, E)], sem))
        for c in copies:
            c.wait()

    return k(patches, table)


# speedup vs baseline: 34.0866x; 34.0866x over previous
import functools

import jax
import jax.numpy as jnp
from jax import lax
from jax.experimental import pallas as pl
from jax.experimental.pallas import tpu as pltpu
from jax.experimental.pallas import tpu_sc as plsc


def kernel(patches, table):
    B, P, D = patches.shape
    E = table.shape[1]
    info = plsc.get_sparse_core_info()
    NC, NS = info.num_cores, info.num_subcores
    NW = NC * NS
    nb = B // NW
    R = 32
    nchunks = P // R
    pairs = [(i, c) for i in range(nb) for c in range(nchunks)]
    mesh = plsc.VectorSubcoreMesh(core_axis_name="c", subcore_axis_name="s")

    @functools.partial(
        pl.kernel,
        mesh=mesh,
        out_type=jax.ShapeDtypeStruct((B, P, D + E), patches.dtype),
        scratch_types=[
            pltpu.VMEM((R, D + E), jnp.float32),
            pltpu.VMEM((R, D + E), jnp.float32),
            pltpu.SemaphoreType.DMA,
            pltpu.SemaphoreType.DMA,
            pltpu.SemaphoreType.DMA,
            pltpu.SemaphoreType.DMA,
        ],
    )
    def k(p_hbm, t_hbm, o_hbm, buf0, buf1, sf0, sf1, sd0, sd1):
        wid = lax.axis_index("s") * NC + lax.axis_index("c")
        bufs = (buf0, buf1)
        sfs = (sf0, sf1)
        sds = (sd0, sd1)
        n = len(pairs)
        fill_h = [None] * n
        drain_h = [None] * n

        def issue_fill(s):
            i, c = pairs[s]
            b = wid * nb + i
            r0 = c * R
            kk = s % 2
            h1 = pltpu.async_copy(
                p_hbm.at[b, pl.ds(r0, R), :], bufs[kk].at[:, pl.ds(0, D)], sfs[kk])
            h2 = pltpu.async_copy(
                t_hbm.at[pl.ds(r0, R), :], bufs[kk].at[:, pl.ds(D, E)], sfs[kk])
            fill_h[s] = (h1, h2)

        def issue_drain(s):
            i, c = pairs[s]
            b = wid * nb + i
            r0 = c * R
            kk = s % 2
            drain_h[s] = pltpu.async_copy(
                bufs[kk], o_hbm.at[b, pl.ds(r0, R), :], sds[kk])

        for s in range(n):
            if s >= 2:
                drain_h[s - 2].wait()
            issue_fill(s)
            if s >= 1:
                for h in fill_h[s - 1]:
                    h.wait()
                issue_drain(s - 1)
        for h in fill_h[n - 1]:
            h.wait()
        issue_drain(n - 1)
        drain_h[n - 2].wait()
        drain_h[n - 1].wait()

    return k(patches, table)


# speedup vs baseline: 63.5646x; 1.8648x over previous
import jax
import jax.numpy as jnp
from jax.experimental import pallas as pl


def _body(p_ref, t_ref, o_ref):
    D = p_ref.shape[-1]
    o_ref[:, :, :D] = p_ref[...]
    o_ref[:, :, D:] = jnp.broadcast_to(t_ref[...][None], o_ref[:, :, D:].shape)


def kernel(patches, table):
    B, P, D = patches.shape
    E = table.shape[1]
    BB = 2
    return pl.pallas_call(
        _body,
        grid=(B // BB,),
        in_specs=[
            pl.BlockSpec((BB, P, D), lambda b: (b, 0, 0)),
            pl.BlockSpec((P, E), lambda b: (0, 0)),
        ],
        out_specs=pl.BlockSpec((BB, P, D + E), lambda b: (b, 0, 0)),
        out_shape=jax.ShapeDtypeStruct((B, P, D + E), patches.dtype),
    )(patches, table)


# speedup vs baseline: 65.9732x; 1.0379x over previous
import jax
import jax.numpy as jnp
from jax.experimental import pallas as pl


def _body(p_ref, t_ref, o_ref):
    D = p_ref.shape[-1]
    o_ref[:, :, :D] = p_ref[...]
    o_ref[:, :, D:] = jnp.broadcast_to(t_ref[...][None], o_ref[:, :, D:].shape)


def kernel(patches, table):
    B, P, D = patches.shape
    E = table.shape[1]
    BB = 4
    return pl.pallas_call(
        _body,
        grid=(B // BB,),
        in_specs=[
            pl.BlockSpec((BB, P, D), lambda b: (b, 0, 0)),
            pl.BlockSpec((P, E), lambda b: (0, 0)),
        ],
        out_specs=pl.BlockSpec((BB, P, D + E), lambda b: (b, 0, 0)),
        out_shape=jax.ShapeDtypeStruct((B, P, D + E), patches.dtype),
    )(patches, table)
